# baseline (speedup 1.0000x reference)
import functools

import jax
import jax.numpy as jnp
from jax import lax
from jax.experimental import pallas as pl
from jax.experimental.pallas import tpu as pltpu
from jax.experimental.pallas import tpu_sc as plsc

N = 10000
E = 160000
D = 256
HD = 128
NS = 16
CHUNK = 128
NBUF = 2
C = 80
CQ = 16
EPS = NS * C * CHUNK
RS = 320
PH = NS * RS
NP = 2 * PH
GARB = PH
ACC_ROWS = PH + 8


def _sc_agg_body(h_hbm, src_hbm, dst_hbm, cnt_hbm, out_hbm, sidx, didx,
                 rowbufs, cbuf, srct, accum, gsems, ssems):
    cid = lax.axis_index("c")
    sid = lax.axis_index("s")
    pltpu.sync_copy(cnt_hbm, cbuf)
    base = sid * RS
    hrow0 = cid * NP

    def gather(j, b):
        pltpu.async_copy(srct.at[sidx.at[j]], rowbufs.at[b], gsems[b])

    def scat(j, b):
        pltpu.async_copy(rowbufs.at[b], accum.at[didx.at[j]],
                         ssems[b], add=True)

    def wait(sem, b):
        pltpu.make_async_copy(h_hbm.at[pl.ds(0, CHUNK)], rowbufs.at[b],
                              sem).wait()

    for d in range(2):
        pltpu.sync_copy(h_hbm.at[pl.ds(hrow0 + d * PH + base, RS)],
                        accum.at[pl.ds(base, RS)])
        for a in range(2):
            pltpu.sync_copy(h_hbm.at[pl.ds(hrow0 + a * PH + base, RS)],
                            srct.at[pl.ds(base, RS)])
            plsc.subcore_barrier()
            bkt = d * 2 + a
            cnt = cbuf[bkt, pl.ds(0, 16)][0]
            nchunks = (cnt + CHUNK - 1) // CHUNK
            m = (nchunks - sid + NS - 1) // NS
            for q in range(C // CQ):
                act = jnp.clip(m - q * CQ, 0, CQ)
                trips = jnp.maximum((act + NBUF - 1) // NBUF, 1)
                pltpu.sync_copy(src_hbm.at[bkt, sid, pl.ds(q * CQ, CQ)],
                                sidx)
                pltpu.sync_copy(dst_hbm.at[bkt, sid, pl.ds(q * CQ, CQ)],
                                didx)
                gather(0, 0)
                gather(1, 1)

                def ring(it, carry):
                    g = it * NBUF
                    wait(gsems[0], 0)
                    scat(g, 0)
                    wait(gsems[1], 1)
                    scat(g + 1, 1)
                    wait(ssems[0], 0)
                    gather(g + NBUF, 0)
                    wait(ssems[1], 1)
                    gather(g + NBUF + 1, 1)
                    return carry

                lax.fori_loop(0, trips - 1, ring, 0, unroll=False)
                g = (trips - 1) * NBUF
                for b in range(NBUF):
                    wait(gsems[b], b)
                    scat(g + b, b)
                for b in range(NBUF):
                    wait(ssems[b], b)
            plsc.subcore_barrier()
        pltpu.sync_copy(accum.at[pl.ds(base, RS)],
                        out_hbm.at[pl.ds(hrow0 + d * PH + base, RS)])
        plsc.subcore_barrier()


_sc_agg = functools.partial(
    pl.kernel,
    out_type=jax.ShapeDtypeStruct((2 * NP, HD), jnp.float32),
    mesh=plsc.VectorSubcoreMesh(core_axis_name="c", subcore_axis_name="s"),
    scratch_types=[
        pltpu.VMEM((CQ, CHUNK), jnp.int32),
        pltpu.VMEM((CQ, CHUNK), jnp.int32),
        pltpu.VMEM((NBUF, CHUNK, HD), jnp.float32),
        pltpu.VMEM((4, 128), jnp.int32),
        pltpu.VMEM_SHARED((PH, HD), jnp.float32),
        pltpu.VMEM_SHARED((ACC_ROWS, HD), jnp.float32),
        [pltpu.SemaphoreType.DMA] * NBUF,
        [pltpu.SemaphoreType.DMA] * NBUF,
    ],
)(_sc_agg_body)


def _bn_relu(h, g, b):
    mu = jnp.mean(h, axis=0, keepdims=True)
    var = jnp.mean((h - mu) * (h - mu), axis=0, keepdims=True)
    h = (h - mu) * lax.rsqrt(var + 1e-5) * g + b
    return jnp.maximum(h, 0.0)


def _tc_mlp_body(s_ref, wa_ref, ba_ref, ga_ref, bea_ref, wb_ref, bb_ref,
                 gb_ref, beb_ref, out_ref):
    s = jnp.concatenate([s_ref[:N], s_ref[NP:NP + N]], axis=1)
    h = jnp.dot(s, wa_ref[...], preferred_element_type=jnp.float32) + ba_ref[...]
    h = _bn_relu(h, ga_ref[...], bea_ref[...])
    h = jnp.dot(h, wb_ref[...], preferred_element_type=jnp.float32) + bb_ref[...]
    h = _bn_relu(h, gb_ref[...], beb_ref[...])
    out_ref[:N] = h[:, :HD]
    out_ref[NP:NP + N] = h[:, HD:]
    out_ref[pl.ds(N, NP - N)] = jnp.zeros((NP - N, HD), jnp.float32)
    out_ref[pl.ds(NP + N, NP - N)] = jnp.zeros((NP - N, HD), jnp.float32)


_tc_mlp = pl.pallas_call(
    _tc_mlp_body,
    out_shape=jax.ShapeDtypeStruct((2 * NP, HD), jnp.float32),
)


def _tc_final_body(s_ref, w_ref, b_ref, g_ref, be_ref, out_ref):
    s = jnp.concatenate([s_ref[:N], s_ref[NP:NP + N]], axis=1)
    h = jnp.dot(s, w_ref[...], preferred_element_type=jnp.float32) + b_ref[...]
    out_ref[...] = _bn_relu(h, g_ref[...], be_ref[...])


_tc_final = pl.pallas_call(
    _tc_final_body,
    out_shape=jax.ShapeDtypeStruct((N, D), jnp.float32),
)


def kernel(x, params, edge_index):
    src = edge_index[0]
    dst = edge_index[1]
    ah = (src >= PH).astype(jnp.int32)
    dh = (dst >= PH).astype(jnp.int32)
    key = dh * 2 + ah
    pos = jnp.zeros((E,), jnp.int32)
    cnts = []
    for b in range(4):
        mb = (key == b).astype(jnp.int32)
        rank = jnp.cumsum(mb) - mb
        pos = pos + mb * (b * EPS + rank)
        cnts.append(jnp.sum(mb))
    srcbuf = jnp.zeros((4 * EPS,), jnp.int32).at[pos].set(src - ah * PH)
    dstbuf = jnp.full((4 * EPS,), GARB, jnp.int32).at[pos].set(dst - dh * PH)
    cnt_hbm = jnp.repeat(jnp.stack(cnts).astype(jnp.int32)[:, None], 128,
                         axis=1)
    src3 = srcbuf.reshape(4, C, NS, CHUNK).transpose(0, 2, 1, 3)
    dst3 = dstbuf.reshape(4, C, NS, CHUNK).transpose(0, 2, 1, 3)

    padrows = jnp.zeros((NP - N, HD), jnp.float32)
    h = jnp.concatenate([x[:, :HD], padrows, x[:, HD:], padrows], axis=0)

    def p2(name):
        return params[name].reshape(1, -1)

    for i in range(2):
        s = _sc_agg(h, src3, dst3, cnt_hbm)
        h = _tc_mlp(s, params['W%da' % i], p2('b%da' % i), p2('g%da' % i),
                    p2('be%da' % i), params['W%db' % i], p2('b%db' % i),
                    p2('g%db' % i), p2('be%db' % i))
    s = _sc_agg(h, src3, dst3, cnt_hbm)
    return _tc_final(s, params['W2'], p2('b2'), p2('g2'), p2('be2'))


# speedup vs baseline: 2.1128x; 2.1128x over previous
import functools

import jax
import jax.numpy as jnp
from jax import lax
from jax.experimental import pallas as pl
from jax.experimental.pallas import tpu as pltpu
from jax.experimental.pallas import tpu_sc as plsc

N = 10000
E = 160000
D = 256
HD = 128
NS = 16
CHUNK = 64
NBUF = 4
C = 160
CQ = 40
EPS = NS * C * CHUNK
ROWS_PER_SUB = 632
NP = NS * ROWS_PER_SUB
GARBAGE = N


def _sc_agg_body(h_hbm, src_hbm, dst_hbm, out_hbm, sidx, didx, rowbufs,
                 shared, gsems, ssems):
    cid = lax.axis_index("c")
    sid = lax.axis_index("s")
    base = sid * ROWS_PER_SUB
    hbase = cid * NP + base
    pltpu.sync_copy(h_hbm.at[pl.ds(hbase, ROWS_PER_SUB)],
                    shared.at[pl.ds(base, ROWS_PER_SUB)])
    plsc.subcore_barrier()

    def gather(j, b):
        pltpu.async_copy(h_hbm.at[sidx.at[j]], rowbufs.at[b], gsems[b])

    def scat(j, b):
        pltpu.async_copy(rowbufs.at[b], shared.at[didx.at[j]],
                         ssems[b], add=True)

    def wait(sem, b):
        pltpu.make_async_copy(h_hbm.at[pl.ds(0, CHUNK)], rowbufs.at[b],
                              sem).wait()

    for quarter in range(C // CQ):
        pltpu.sync_copy(src_hbm.at[cid, sid, pl.ds(quarter * CQ, CQ)], sidx)
        pltpu.sync_copy(dst_hbm.at[sid, pl.ds(quarter * CQ, CQ)], didx)
        for b in range(NBUF):
            gather(b, b)

        def ring(it, carry):
            g = it * NBUF
            for b in range(NBUF):
                wait(gsems[b], b)
                scat(g + b, b)
            for b in range(NBUF):
                wait(ssems[b], b)
                gather(g + NBUF + b, b)
            return carry

        lax.fori_loop(0, CQ // NBUF - 1, ring, 0, unroll=False)
        g = CQ - NBUF
        for b in range(NBUF):
            wait(gsems[b], b)
            scat(g + b, b)
        for b in range(NBUF):
            wait(ssems[b], b)
    plsc.subcore_barrier()
    pltpu.sync_copy(shared.at[pl.ds(base, ROWS_PER_SUB)],
                    out_hbm.at[pl.ds(hbase, ROWS_PER_SUB)])


_sc_agg = functools.partial(
    pl.kernel,
    out_type=jax.ShapeDtypeStruct((2 * NP, HD), jnp.float32),
    mesh=plsc.VectorSubcoreMesh(core_axis_name="c", subcore_axis_name="s"),
    scratch_types=[
        pltpu.VMEM((CQ, CHUNK), jnp.int32),
        pltpu.VMEM((CQ, CHUNK), jnp.int32),
        pltpu.VMEM((NBUF, CHUNK, HD), jnp.float32),
        pltpu.VMEM_SHARED((NP + 8, HD), jnp.float32),
        [pltpu.SemaphoreType.DMA] * NBUF,
        [pltpu.SemaphoreType.DMA] * NBUF,
    ],
)(_sc_agg_body)


def _bn_relu(h, g, b):
    mu = jnp.mean(h, axis=0, keepdims=True)
    var = jnp.mean((h - mu) * (h - mu), axis=0, keepdims=True)
    h = (h - mu) * lax.rsqrt(var + 1e-5) * g + b
    return jnp.maximum(h, 0.0)


def _tc_mlp_body(s_ref, wa_ref, ba_ref, ga_ref, bea_ref, wb_ref, bb_ref,
                 gb_ref, beb_ref, out_ref):
    s = jnp.concatenate([s_ref[:N], s_ref[NP:NP + N]], axis=1)
    h = jnp.dot(s, wa_ref[...], preferred_element_type=jnp.float32) + ba_ref[...]
    h = _bn_relu(h, ga_ref[...], bea_ref[...])
    h = jnp.dot(h, wb_ref[...], preferred_element_type=jnp.float32) + bb_ref[...]
    h = _bn_relu(h, gb_ref[...], beb_ref[...])
    out_ref[:N] = h[:, :HD]
    out_ref[NP:NP + N] = h[:, HD:]


_tc_mlp = pl.pallas_call(
    _tc_mlp_body,
    out_shape=jax.ShapeDtypeStruct((2 * NP, HD), jnp.float32),
)


def _tc_final_body(s_ref, w_ref, b_ref, g_ref, be_ref, out_ref):
    s = jnp.concatenate([s_ref[:N], s_ref[NP:NP + N]], axis=1)
    h = jnp.dot(s, w_ref[...], preferred_element_type=jnp.float32) + b_ref[...]
    out_ref[...] = _bn_relu(h, g_ref[...], be_ref[...])


_tc_final = pl.pallas_call(
    _tc_final_body,
    out_shape=jax.ShapeDtypeStruct((N, D), jnp.float32),
)


def kernel(x, params, edge_index):
    src = edge_index[0]
    dst = edge_index[1]
    pad = EPS - E
    src_p = jnp.concatenate([src, jnp.zeros((pad,), jnp.int32)])
    dst_p = jnp.concatenate([dst, jnp.full((pad,), GARBAGE, jnp.int32)])
    src3 = (src_p[None, :] + (jnp.arange(2, dtype=jnp.int32) * NP)[:, None]
            ).reshape(2, NS, C, CHUNK)
    dst2 = dst_p.reshape(NS, C, CHUNK)

    padrows = jnp.zeros((NP - N, HD), jnp.float32)
    h = jnp.concatenate([x[:, :HD], padrows, x[:, HD:], padrows], axis=0)

    def p2(name):
        return params[name].reshape(1, -1)

    for i in range(2):
        s = _sc_agg(h, src3, dst2)
        h = _tc_mlp(s, params['W%da' % i], p2('b%da' % i), p2('g%da' % i),
                    p2('be%da' % i), params['W%db' % i], p2('b%db' % i),
                    p2('g%db' % i), p2('be%db' % i))
    s = _sc_agg(h, src3, dst2)
    return _tc_final(s, params['W2'], p2('b2'), p2('g2'), p2('be2'))
